# baseline (speedup 1.0000x reference)
import functools

import jax
import jax.numpy as jnp
from jax import lax
from jax.experimental import pallas as pl
from jax.experimental.pallas import tpu as pltpu
from jax.experimental.pallas import tpu_sc as plsc

N = 10000
NP = 10240
E = 320000
DUMMY = N
LDIMS = [(128, 85), (85, 56), (56, 28), (28, 1)]
WIN = [128, 96, 64, 32]
WOUT = [96, 64, 32, 16]
NW = 32
CHUNK = 128
NCH = 80
EPT = NCH * CHUNK
EP = NW * EPT
RPT = NP // 16
TCB = 1024
K0 = 64
K1 = 160 - K0


def _make_sc_seg_sum(wp, k0, k1):
  mesh = plsc.VectorSubcoreMesh(core_axis_name="c", subcore_axis_name="s")
  kmax = max(k0, k1)

  @functools.partial(
      pl.kernel,
      mesh=mesh,
      compiler_params=pltpu.CompilerParams(use_tc_tiling_on_sc=False),
      out_type=(
          jax.ShapeDtypeStruct((NP, wp), jnp.float32),
          jax.ShapeDtypeStruct((NP, wp), jnp.float32),
      ),
      scratch_types=[
          pltpu.VMEM((kmax, CHUNK), jnp.int32),
          pltpu.VMEM((kmax, CHUNK), jnp.int32),
          pltpu.VMEM((CHUNK, wp), jnp.float32),
          pltpu.VMEM_SHARED((NP, wp), jnp.float32),
          pltpu.SemaphoreType.DMA,
      ],
  )
  def seg_sum(p_hbm, src0_hbm, dst0_hbm, src1_hbm, dst1_hbm, zero_hbm,
              out_a, out_b, src_v, dst_v, rows_v, acc, sem):
    c = lax.axis_index("c")
    s = lax.axis_index("s")
    r0 = s * RPT

    @pl.when(c == 0)
    def _():
      pltpu.sync_copy(src0_hbm.at[s], src_v.at[pl.ds(0, k0)])
      pltpu.sync_copy(dst0_hbm.at[s], dst_v.at[pl.ds(0, k0)])

    @pl.when(c == 1)
    def _():
      pltpu.sync_copy(src1_hbm.at[s], src_v.at[pl.ds(0, k1)])
      pltpu.sync_copy(dst1_hbm.at[s], dst_v.at[pl.ds(0, k1)])

    pltpu.sync_copy(zero_hbm.at[pl.ds(r0, RPT)], acc.at[pl.ds(r0, RPT)])
    plsc.subcore_barrier()

    def body(j, carry):
      pltpu.async_copy(p_hbm.at[src_v.at[j]], rows_v, sem).wait()
      pltpu.sync_copy(rows_v, acc.at[dst_v.at[j]], add=True)
      return carry

    nch = jnp.where(c == 0, k0, k1)
    lax.fori_loop(0, nch, body, 0)
    plsc.subcore_barrier()

    @pl.when(c == 0)
    def _():
      pltpu.sync_copy(acc.at[pl.ds(r0, RPT)], out_a.at[pl.ds(r0, RPT)])

    @pl.when(c == 1)
    def _():
      pltpu.sync_copy(acc.at[pl.ds(r0, RPT)], out_b.at[pl.ds(r0, RPT)])

  return seg_sum


def _mm_body(x_ref, wl_ref, wr_ref, cv_ref, bv_ref, p_ref, q_ref):
  x = x_ref[...]
  p_ref[...] = jnp.dot(x, wl_ref[...],
                       preferred_element_type=jnp.float32) + cv_ref[...]
  q_ref[...] = jnp.dot(x, wr_ref[...],
                       preferred_element_type=jnp.float32) + bv_ref[...]


def _comb_body(dcol, sa_ref, sb_ref, qp_ref, wl_ref, wr_ref, cv_ref, bv_ref,
               p_ref, q_ref):
  ssum = sa_ref[...] + sb_ref[...]
  cnt = ssum[:, dcol:dcol + 1]
  inv = 1.0 / jnp.maximum(cnt, 1.0)
  h = ssum * inv + qp_ref[...]
  p_ref[...] = jnp.dot(h, wl_ref[...],
                       preferred_element_type=jnp.float32) + cv_ref[...]
  q_ref[...] = jnp.dot(h, wr_ref[...],
                       preferred_element_type=jnp.float32) + bv_ref[...]


def _final_body(sa_ref, sb_ref, qp_ref, mask_ref, out_ref):
  ssum = sa_ref[...] + sb_ref[...]
  cnt = ssum[:, 1:2]
  inv = 1.0 / jnp.maximum(cnt, 1.0)
  out_ref[...] = (ssum * inv + qp_ref[...]) * mask_ref[...]


def _row_spec(w):
  return pl.BlockSpec((TCB, w), lambda i: (i, 0))


def _full_spec(r, cm):
  return pl.BlockSpec((r, cm), lambda i: (0, 0))


def _transform_call(x, wl, wr, cv, bv):
  win, wout = wl.shape
  return pl.pallas_call(
      _mm_body,
      grid=(NP // TCB,),
      in_specs=[
          _row_spec(win),
          _full_spec(win, wout),
          _full_spec(win, wout),
          _full_spec(1, wout),
          _full_spec(1, wout),
      ],
      out_specs=[_row_spec(wout), _row_spec(wout)],
      out_shape=[jax.ShapeDtypeStruct((NP, wout), jnp.float32)] * 2,
  )(x, wl, wr, cv, bv)


def _combine_call(sa, sb, qp, wl, wr, cv, bv, dcol):
  win, wout = wl.shape
  return pl.pallas_call(
      functools.partial(_comb_body, dcol),
      grid=(NP // TCB,),
      in_specs=[
          _row_spec(win),
          _row_spec(win),
          _row_spec(win),
          _full_spec(win, wout),
          _full_spec(win, wout),
          _full_spec(1, wout),
          _full_spec(1, wout),
      ],
      out_specs=[_row_spec(wout), _row_spec(wout)],
      out_shape=[jax.ShapeDtypeStruct((NP, wout), jnp.float32)] * 2,
  )(sa, sb, qp, wl, wr, cv, bv)


def _final_call(sa, sb, qp, mask):
  w = sa.shape[1]
  return pl.pallas_call(
      _final_body,
      grid=(NP // TCB,),
      in_specs=[
          _row_spec(w),
          _row_spec(w),
          _row_spec(w),
          _full_spec(1, w),
      ],
      out_specs=_row_spec(w),
      out_shape=jax.ShapeDtypeStruct((NP, w), jnp.float32),
  )(sa, sb, qp, mask)


def kernel(x, edge_index, Wl0, bl0, Wr0, Wl1, bl1, Wr1, Wl2, bl2, Wr2,
           Wl3, bl3, Wr3):
  f32 = jnp.float32
  ei = edge_index.astype(jnp.int32)
  pad_e = EP - E
  pad_idx = jnp.full((pad_e,), DUMMY, jnp.int32)
  srcf = jnp.concatenate([ei[0], pad_idx]).reshape(-1, CHUNK)
  dstf = jnp.concatenate([ei[1], pad_idx]).reshape(-1, CHUNK)
  src0 = srcf[:16 * K0].reshape(16, K0, CHUNK)
  src1 = srcf[16 * K0:].reshape(16, K1, CHUNK)
  dst0 = dstf[:16 * K0].reshape(16, K0, CHUNK)
  dst1 = dstf[16 * K0:].reshape(16, K1, CHUNK)
  xp = jnp.zeros((NP, 128), f32).at[:N].set(x)

  wls = [Wl0, Wl1, Wl2, Wl3]
  bls = [bl0, bl1, bl2, bl3]
  wrs = [Wr0, Wr1, Wr2, Wr3]
  wl_p, wr_p, cv_p, bv_p = [], [], [], []
  for l, (din, dout) in enumerate(LDIMS):
    win, wout = WIN[l], WOUT[l]
    wl_p.append(jnp.zeros((win, wout), f32).at[:din, :dout].set(wls[l].T))
    wr_p.append(jnp.zeros((win, wout), f32).at[:din, :dout].set(wrs[l].T))
    cv_p.append(jnp.zeros((1, wout), f32).at[0, dout].set(1.0))
    bv_p.append(jnp.zeros((1, wout), f32).at[0, :dout].set(bls[l]))

  sc_calls = [_make_sc_seg_sum(w, K0, K1) for w in WOUT]

  p, q = _transform_call(xp, wl_p[0], wr_p[0], cv_p[0], bv_p[0])
  out = None
  for l in range(4):
    zeros_l = jnp.zeros((NP, WOUT[l]), f32)
    sa, sb = sc_calls[l](p, src0, dst0, src1, dst1, zeros_l)
    if l < 3:
      p, q = _combine_call(sa, sb, q, wl_p[l + 1], wr_p[l + 1],
                           cv_p[l + 1], bv_p[l + 1], LDIMS[l + 1][0])
    else:
      mask = jnp.zeros((1, WOUT[3]), f32).at[0, 0].set(1.0)
      out = _final_call(sa, sb, q, mask)
  return out[:N, 0:1]


# speedup vs baseline: 1.2463x; 1.2463x over previous
import functools

import jax
import jax.numpy as jnp
from jax import lax
from jax.experimental import pallas as pl
from jax.experimental.pallas import tpu as pltpu
from jax.experimental.pallas import tpu_sc as plsc

N = 10000
NP = 10240
E = 320000
DUMMY = N
LDIMS = [(128, 85), (85, 56), (56, 28), (28, 1)]
WIN = [128, 96, 64, 32]
WOUT = [96, 64, 32, 16]
NW = 32
CHUNK = 128
NCH = 80
EPT = NCH * CHUNK
EP = NW * EPT
RPT = NP // 16
TCB = 1024
K0 = 80
K1 = 160 - K0


def _make_sc_seg_sum(wp, k0, k1):
  mesh = plsc.VectorSubcoreMesh(core_axis_name="c", subcore_axis_name="s")
  kmax = max(k0, k1)

  @functools.partial(
      pl.kernel,
      mesh=mesh,
      compiler_params=pltpu.CompilerParams(use_tc_tiling_on_sc=False),
      out_type=(
          jax.ShapeDtypeStruct((NP, wp), jnp.float32),
          jax.ShapeDtypeStruct((NP, wp), jnp.float32),
      ),
      scratch_types=[
          pltpu.VMEM((kmax, CHUNK), jnp.int32),
          pltpu.VMEM((kmax, CHUNK), jnp.int32),
          pltpu.VMEM((CHUNK, wp), jnp.float32),
          pltpu.VMEM((CHUNK, wp), jnp.float32),
          pltpu.VMEM_SHARED((NP, wp), jnp.float32),
          pltpu.SemaphoreType.DMA,
      ],
  )
  def seg_sum(p_hbm, src0_hbm, dst0_hbm, src1_hbm, dst1_hbm, zero_hbm,
              out_a, out_b, src_v, dst_v, rows0_v, rows1_v, acc, sem):
    c = lax.axis_index("c")
    s = lax.axis_index("s")
    r0 = s * RPT

    @pl.when(c == 0)
    def _():
      pltpu.sync_copy(src0_hbm.at[s], src_v.at[pl.ds(0, k0)])
      pltpu.sync_copy(dst0_hbm.at[s], dst_v.at[pl.ds(0, k0)])

    @pl.when(c == 1)
    def _():
      pltpu.sync_copy(src1_hbm.at[s], src_v.at[pl.ds(0, k1)])
      pltpu.sync_copy(dst1_hbm.at[s], dst_v.at[pl.ds(0, k1)])

    pltpu.sync_copy(zero_hbm.at[pl.ds(r0, RPT)], acc.at[pl.ds(r0, RPT)])
    plsc.subcore_barrier()

    nch = jnp.where(c == 0, k0, k1)
    pltpu.async_copy(p_hbm.at[src_v.at[0]], rows0_v, sem)

    def body(i, carry):
      j = 2 * i
      pltpu.make_async_copy(p_hbm.at[src_v.at[j]], rows0_v, sem).wait()
      pltpu.async_copy(p_hbm.at[src_v.at[jnp.minimum(j + 1, nch - 1)]],
                       rows1_v, sem)
      pltpu.sync_copy(rows0_v, acc.at[dst_v.at[j]], add=True)
      pltpu.make_async_copy(p_hbm.at[src_v.at[j + 1]], rows1_v, sem).wait()
      pltpu.async_copy(p_hbm.at[src_v.at[jnp.minimum(j + 2, nch - 1)]],
                       rows0_v, sem)
      pltpu.sync_copy(rows1_v, acc.at[dst_v.at[j + 1]], add=True)
      return carry

    lax.fori_loop(0, nch // 2, body, 0)
    pltpu.make_async_copy(p_hbm.at[src_v.at[nch - 1]], rows0_v, sem).wait()
    plsc.subcore_barrier()

    @pl.when(c == 0)
    def _():
      pltpu.sync_copy(acc.at[pl.ds(r0, RPT)], out_a.at[pl.ds(r0, RPT)])

    @pl.when(c == 1)
    def _():
      pltpu.sync_copy(acc.at[pl.ds(r0, RPT)], out_b.at[pl.ds(r0, RPT)])

  return seg_sum


def _mm_body(x_ref, wl_ref, wr_ref, cv_ref, bv_ref, p_ref, q_ref):
  x = x_ref[...]
  p_ref[...] = jnp.dot(x, wl_ref[...],
                       preferred_element_type=jnp.float32) + cv_ref[...]
  q_ref[...] = jnp.dot(x, wr_ref[...],
                       preferred_element_type=jnp.float32) + bv_ref[...]


def _comb_body(dcol, sa_ref, sb_ref, qp_ref, wl_ref, wr_ref, cv_ref, bv_ref,
               p_ref, q_ref):
  ssum = sa_ref[...] + sb_ref[...]
  cnt = ssum[:, dcol:dcol + 1]
  inv = 1.0 / jnp.maximum(cnt, 1.0)
  h = ssum * inv + qp_ref[...]
  p_ref[...] = jnp.dot(h, wl_ref[...],
                       preferred_element_type=jnp.float32) + cv_ref[...]
  q_ref[...] = jnp.dot(h, wr_ref[...],
                       preferred_element_type=jnp.float32) + bv_ref[...]


def _final_body(sa_ref, sb_ref, qp_ref, mask_ref, out_ref):
  ssum = sa_ref[...] + sb_ref[...]
  cnt = ssum[:, 1:2]
  inv = 1.0 / jnp.maximum(cnt, 1.0)
  out_ref[...] = (ssum * inv + qp_ref[...]) * mask_ref[...]


def _row_spec(w):
  return pl.BlockSpec((TCB, w), lambda i: (i, 0))


def _full_spec(r, cm):
  return pl.BlockSpec((r, cm), lambda i: (0, 0))


def _transform_call(x, wl, wr, cv, bv):
  win, wout = wl.shape
  return pl.pallas_call(
      _mm_body,
      grid=(NP // TCB,),
      in_specs=[
          _row_spec(win),
          _full_spec(win, wout),
          _full_spec(win, wout),
          _full_spec(1, wout),
          _full_spec(1, wout),
      ],
      out_specs=[_row_spec(wout), _row_spec(wout)],
      out_shape=[jax.ShapeDtypeStruct((NP, wout), jnp.float32)] * 2,
  )(x, wl, wr, cv, bv)


def _combine_call(sa, sb, qp, wl, wr, cv, bv, dcol):
  win, wout = wl.shape
  return pl.pallas_call(
      functools.partial(_comb_body, dcol),
      grid=(NP // TCB,),
      in_specs=[
          _row_spec(win),
          _row_spec(win),
          _row_spec(win),
          _full_spec(win, wout),
          _full_spec(win, wout),
          _full_spec(1, wout),
          _full_spec(1, wout),
      ],
      out_specs=[_row_spec(wout), _row_spec(wout)],
      out_shape=[jax.ShapeDtypeStruct((NP, wout), jnp.float32)] * 2,
  )(sa, sb, qp, wl, wr, cv, bv)


def _final_call(sa, sb, qp, mask):
  w = sa.shape[1]
  return pl.pallas_call(
      _final_body,
      grid=(NP // TCB,),
      in_specs=[
          _row_spec(w),
          _row_spec(w),
          _row_spec(w),
          _full_spec(1, w),
      ],
      out_specs=_row_spec(w),
      out_shape=jax.ShapeDtypeStruct((NP, w), jnp.float32),
  )(sa, sb, qp, mask)


def kernel(x, edge_index, Wl0, bl0, Wr0, Wl1, bl1, Wr1, Wl2, bl2, Wr2,
           Wl3, bl3, Wr3):
  f32 = jnp.float32
  ei = edge_index.astype(jnp.int32)
  pad_e = EP - E
  pad_idx = jnp.full((pad_e,), DUMMY, jnp.int32)
  srcf = jnp.concatenate([ei[0], pad_idx]).reshape(-1, CHUNK)
  dstf = jnp.concatenate([ei[1], pad_idx]).reshape(-1, CHUNK)
  src0 = srcf[:16 * K0].reshape(16, K0, CHUNK)
  src1 = srcf[16 * K0:].reshape(16, K1, CHUNK)
  dst0 = dstf[:16 * K0].reshape(16, K0, CHUNK)
  dst1 = dstf[16 * K0:].reshape(16, K1, CHUNK)
  xp = jnp.zeros((NP, 128), f32).at[:N].set(x)

  wls = [Wl0, Wl1, Wl2, Wl3]
  bls = [bl0, bl1, bl2, bl3]
  wrs = [Wr0, Wr1, Wr2, Wr3]
  wl_p, wr_p, cv_p, bv_p = [], [], [], []
  for l, (din, dout) in enumerate(LDIMS):
    win, wout = WIN[l], WOUT[l]
    wl_p.append(jnp.zeros((win, wout), f32).at[:din, :dout].set(wls[l].T))
    wr_p.append(jnp.zeros((win, wout), f32).at[:din, :dout].set(wrs[l].T))
    cv_p.append(jnp.zeros((1, wout), f32).at[0, dout].set(1.0))
    bv_p.append(jnp.zeros((1, wout), f32).at[0, :dout].set(bls[l]))

  sc_calls = [_make_sc_seg_sum(w, K0, K1) for w in WOUT]

  p, q = _transform_call(xp, wl_p[0], wr_p[0], cv_p[0], bv_p[0])
  out = None
  for l in range(4):
    zeros_l = jnp.zeros((NP, WOUT[l]), f32)
    sa, sb = sc_calls[l](p, src0, dst0, src1, dst1, zeros_l)
    if l < 3:
      p, q = _combine_call(sa, sb, q, wl_p[l + 1], wr_p[l + 1],
                           cv_p[l + 1], bv_p[l + 1], LDIMS[l + 1][0])
    else:
      mask = jnp.zeros((1, WOUT[3]), f32).at[0, 0].set(1.0)
      out = _final_call(sa, sb, q, mask)
  return out[:N, 0:1]


# speedup vs baseline: 5.4861x; 4.4018x over previous
import functools

import jax
import jax.numpy as jnp
from jax import lax
from jax.experimental import pallas as pl
from jax.experimental.pallas import tpu as pltpu
from jax.experimental.pallas import tpu_sc as plsc

N = 10000
NP = 10240
E = 320000
DUMMY = N
NW = 32
EPT = 10240
EP = NW * EPT
RPT = NP // 16
TCB = 1024
WA = 144
ONE_COL = 128
LDIMS = [(128, 85), (85, 56), (56, 28), (28, 1)]


@functools.cache
def _make_sc_apply_a():
  mesh = plsc.VectorSubcoreMesh(core_axis_name="c", subcore_axis_name="s")

  @functools.partial(
      pl.kernel,
      mesh=mesh,
      compiler_params=pltpu.CompilerParams(use_tc_tiling_on_sc=False,
                                           needs_layout_passes=False),
      out_type=(
          jax.ShapeDtypeStruct((NP,), jnp.float32),
          jax.ShapeDtypeStruct((NP,), jnp.float32),
          jax.ShapeDtypeStruct((NP,), jnp.float32),
          jax.ShapeDtypeStruct((NP,), jnp.float32),
      ),
      scratch_types=[
          pltpu.VMEM((NP,), jnp.float32),
          pltpu.VMEM((EPT,), jnp.int32),
          pltpu.VMEM((EPT,), jnp.int32),
          pltpu.VMEM((NP,), jnp.float32),
          pltpu.VMEM((NP,), jnp.float32),
          pltpu.VMEM((16, RPT), jnp.float32),
          pltpu.VMEM_SHARED((16, NP), jnp.float32),
          pltpu.VMEM_SHARED((16, NP), jnp.float32),
      ],
  )
  def sc_apply(y_hbm, src_hbm, dst_hbm, sum_a, sum_b, cnt_a, cnt_b,
               pval_v, src_v, dst_v, acc_v, cnt_v, tmp_v, sh_sum, sh_cnt):
    c = lax.axis_index("c")
    s = lax.axis_index("s")
    wid = c * 16 + s
    r0 = s * RPT
    pltpu.sync_copy(y_hbm, pval_v)
    pltpu.sync_copy(src_hbm.at[wid], src_v)
    pltpu.sync_copy(dst_hbm.at[wid], dst_v)

    zeros16 = jnp.zeros((16,), jnp.float32)
    ones16 = jnp.ones((16,), jnp.float32)

    def zbody(i, carry):
      acc_v[pl.ds(16 * i, 16)] = zeros16
      cnt_v[pl.ds(16 * i, 16)] = zeros16
      return carry

    lax.fori_loop(0, NP // 16, zbody, 0)

    def ebody(e, carry):
      si = src_v[pl.ds(16 * e, 16)]
      di = dst_v[pl.ds(16 * e, 16)]
      vals = plsc.load_gather(pval_v, [si])
      plsc.addupdate_scatter(acc_v, [di], vals)
      plsc.addupdate_scatter(cnt_v, [di], ones16)
      return carry

    lax.fori_loop(0, EPT // 16, ebody, 0)

    pltpu.sync_copy(acc_v, sh_sum.at[s])
    pltpu.sync_copy(cnt_v, sh_cnt.at[s])
    plsc.subcore_barrier()

    for sh, out_0, out_1 in ((sh_sum, sum_a, sum_b), (sh_cnt, cnt_a, cnt_b)):
      pltpu.sync_copy(sh.at[:, pl.ds(r0, RPT)], tmp_v)

      def rbody(i, carry):
        tot = tmp_v[0, pl.ds(16 * i, 16)]
        for t in range(1, 16):
          tot = tot + tmp_v[t, pl.ds(16 * i, 16)]
        acc_v[pl.ds(16 * i, 16)] = tot
        return carry

      lax.fori_loop(0, RPT // 16, rbody, 0)

      @pl.when(c == 0)
      def _():
        pltpu.sync_copy(acc_v.at[pl.ds(0, RPT)], out_0.at[pl.ds(r0, RPT)])

      @pl.when(c == 1)
      def _():
        pltpu.sync_copy(acc_v.at[pl.ds(0, RPT)], out_1.at[pl.ds(r0, RPT)])

  return sc_apply


def _sc_apply_a(y, src_t, dst_t):
  return _make_sc_apply_a()(y, src_t, dst_t)


def _prep_body(eye_ref, b0, b1, b2, b3, c0, c1, c2, c3, mpack_ref):
  bs = [b0[...], b1[...], b2[...], b3[...]]
  cs = [c0[...], c1[...], c2[...], c3[...]]
  ms = [eye_ref[...], None, None, None, None]
  for l in range(4):
    new = []
    for k in range(5):
      t = None
      if ms[k] is not None:
        t = jnp.dot(ms[k], cs[l], preferred_element_type=jnp.float32)
      if k > 0 and ms[k - 1] is not None:
        tb = jnp.dot(ms[k - 1], bs[l], preferred_element_type=jnp.float32)
        t = tb if t is None else t + tb
      new.append(t)
    ms = new
  cols = [m[:, 0:1] for m in ms]
  cols.append(jnp.zeros((WA, 128 - 5), jnp.float32))
  mpack_ref[...] = jnp.concatenate(cols, axis=1)


def _z_body(x_ref, mp_ref, z_ref, y_ref):
  z = jnp.dot(x_ref[...], mp_ref[...], preferred_element_type=jnp.float32)
  z_ref[...] = z
  y_ref[...] = z[:, 4]


def _horner_body(k, sa_ref, sb_ref, ca_ref, cb_ref, z_ref, y_ref):
  cnt = ca_ref[...] + cb_ref[...]
  inv = 1.0 / jnp.maximum(cnt, 1.0)
  y_ref[...] = (sa_ref[...] + sb_ref[...]) * inv + z_ref[...][:, k]


def _w_spec():
  return pl.BlockSpec((WA, WA), lambda i: (0, 0))


def _prep_call(eye, bs, cs):
  return pl.pallas_call(
      _prep_body,
      grid=(1,),
      in_specs=[_w_spec()] * 9,
      out_specs=pl.BlockSpec((WA, 128), lambda i: (0, 0)),
      out_shape=jax.ShapeDtypeStruct((WA, 128), jnp.float32),
  )(eye, *bs, *cs)


def _z_call(xp2, mpack):
  return pl.pallas_call(
      _z_body,
      grid=(NP // TCB,),
      in_specs=[
          pl.BlockSpec((TCB, WA), lambda i: (i, 0)),
          pl.BlockSpec((WA, 128), lambda i: (0, 0)),
      ],
      out_specs=[
          pl.BlockSpec((TCB, 128), lambda i: (i, 0)),
          pl.BlockSpec((TCB,), lambda i: (i,)),
      ],
      out_shape=[
          jax.ShapeDtypeStruct((NP, 128), jnp.float32),
          jax.ShapeDtypeStruct((NP,), jnp.float32),
      ],
  )(xp2, mpack)


def _horner_call(sa, sb, ca, cb, z_all, k):
  return pl.pallas_call(
      functools.partial(_horner_body, k),
      grid=(NP // TCB,),
      in_specs=[pl.BlockSpec((TCB,), lambda i: (i,))] * 4
      + [pl.BlockSpec((TCB, 128), lambda i: (i, 0))],
      out_specs=pl.BlockSpec((TCB,), lambda i: (i,)),
      out_shape=jax.ShapeDtypeStruct((NP,), jnp.float32),
  )(sa, sb, ca, cb, z_all)


def kernel(x, edge_index, Wl0, bl0, Wr0, Wl1, bl1, Wr1, Wl2, bl2, Wr2,
           Wl3, bl3, Wr3):
  f32 = jnp.float32
  ei = edge_index.astype(jnp.int32)
  pad_idx = jnp.full((EP - E,), DUMMY, jnp.int32)
  src_t = jnp.concatenate([ei[0], pad_idx]).reshape(NW, EPT)
  dst_t = jnp.concatenate([ei[1], pad_idx]).reshape(NW, EPT)

  xp2 = jnp.zeros((NP, WA), f32).at[:N, :128].set(x).at[:, ONE_COL].set(1.0)

  wls = [Wl0, Wl1, Wl2, Wl3]
  bls = [bl0, bl1, bl2, bl3]
  wrs = [Wr0, Wr1, Wr2, Wr3]
  bs, cs = [], []
  for l, (din, dout) in enumerate(LDIMS):
    bs.append(jnp.zeros((WA, WA), f32).at[:din, :dout].set(wls[l].T))
    cs.append(
        jnp.zeros((WA, WA), f32)
        .at[:din, :dout].set(wrs[l].T)
        .at[ONE_COL, :dout].set(bls[l])
        .at[ONE_COL, ONE_COL].set(1.0)
    )
  eye = jnp.eye(WA, dtype=f32)

  mpack = _prep_call(eye, bs, cs)
  z_all, y = _z_call(xp2, mpack)
  for k in (3, 2, 1, 0):
    sum_a, sum_b, cnt_a, cnt_b = _sc_apply_a(y, src_t, dst_t)
    y = _horner_call(sum_a, sum_b, cnt_a, cnt_b, z_all, k)
  return y[:N].reshape(N, 1)


# speedup vs baseline: 7.2558x; 1.3226x over previous
import functools

import jax
import jax.numpy as jnp
from jax import lax
from jax.experimental import pallas as pl
from jax.experimental.pallas import tpu as pltpu
from jax.experimental.pallas import tpu_sc as plsc

N = 10000
NP = 10240
E = 320000
DUMMY = N
NT = 16
EPT = 20480
EP = NT * EPT
RPT = NP // NT
TCB = 1024
WA = 144
ONE_COL = 128
LDIMS = [(128, 85), (85, 56), (56, 28), (28, 1)]


@functools.cache
def _make_sc_horner():
  mesh = plsc.VectorSubcoreMesh(core_axis_name="c", subcore_axis_name="s")

  @functools.partial(
      pl.kernel,
      mesh=mesh,
      compiler_params=pltpu.CompilerParams(use_tc_tiling_on_sc=False,
                                           needs_layout_passes=False),
      out_type=jax.ShapeDtypeStruct((NP,), jnp.float32),
      scratch_types=[
          pltpu.VMEM((NP,), jnp.float32),
          pltpu.VMEM((EPT,), jnp.int32),
          pltpu.VMEM((EPT,), jnp.int32),
          pltpu.VMEM((NP,), jnp.float32),
          pltpu.VMEM((NP,), jnp.float32),
          pltpu.VMEM((16, RPT), jnp.float32),
          pltpu.VMEM((RPT,), jnp.float32),
          pltpu.VMEM((RPT,), jnp.float32),
          pltpu.VMEM((4, RPT), jnp.float32),
          pltpu.VMEM_SHARED((16, NP), jnp.float32),
          pltpu.VMEM_SHARED((NP,), jnp.float32),
      ],
  )
  def sc_horner(y4_hbm, z3_hbm, z2_hbm, z1_hbm, z0_hbm, src_hbm, dst_hbm,
                out, pval_v, src_v, dst_v, acc_v, cnt_v, tmp_v, inv_v,
                ynew_v, z_v, sh_part, sh_y):
    c = lax.axis_index("c")
    s = lax.axis_index("s")
    r0 = s * RPT

    @pl.when(c == 0)
    def _body():
      pltpu.sync_copy(y4_hbm, pval_v)
      pltpu.sync_copy(src_hbm.at[s], src_v)
      pltpu.sync_copy(dst_hbm.at[s], dst_v)
      for j, z_hbm in enumerate((z3_hbm, z2_hbm, z1_hbm, z0_hbm)):
        pltpu.sync_copy(z_hbm.at[pl.ds(r0, RPT)], z_v.at[j])

      zeros16 = jnp.zeros((16,), jnp.float32)
      ones16 = jnp.ones((16,), jnp.float32)

      for step, k in enumerate((3, 2, 1, 0)):
        first = step == 0

        def zbody(i, carry):
          for u in range(4):
            acc_v[pl.ds(64 * i + 16 * u, 16)] = zeros16
          return carry

        lax.fori_loop(0, NP // 64, zbody, 0)
        if first:

          def czbody(i, carry):
            for u in range(4):
              cnt_v[pl.ds(64 * i + 16 * u, 16)] = zeros16
            return carry

          lax.fori_loop(0, NP // 64, czbody, 0)

        def ebody(e, carry):
          for u in range(2):
            o = 32 * e + 16 * u
            si = src_v[pl.ds(o, 16)]
            di = dst_v[pl.ds(o, 16)]
            vals = plsc.load_gather(pval_v, [si])
            plsc.addupdate_scatter(acc_v, [di], vals)
            if first:
              plsc.addupdate_scatter(cnt_v, [di], ones16)
          return carry

        lax.fori_loop(0, EPT // 32, ebody, 0)

        pltpu.sync_copy(acc_v, sh_part.at[s])
        plsc.subcore_barrier()
        pltpu.sync_copy(sh_part.at[:, pl.ds(r0, RPT)], tmp_v)

        if first:
          def sbody(i, carry):
            tot = tmp_v[0, pl.ds(16 * i, 16)]
            for t in range(1, 16):
              tot = tot + tmp_v[t, pl.ds(16 * i, 16)]
            ynew_v[pl.ds(16 * i, 16)] = tot
            return carry

          lax.fori_loop(0, RPT // 16, sbody, 0)
          plsc.subcore_barrier()
          pltpu.sync_copy(cnt_v, sh_part.at[s])
          plsc.subcore_barrier()
          pltpu.sync_copy(sh_part.at[:, pl.ds(r0, RPT)], tmp_v)

          def ibody(i, carry):
            tot = tmp_v[0, pl.ds(16 * i, 16)]
            for t in range(1, 16):
              tot = tot + tmp_v[t, pl.ds(16 * i, 16)]
            inv_v[pl.ds(16 * i, 16)] = 1.0 / jnp.maximum(tot, 1.0)
            ynew_v[pl.ds(16 * i, 16)] = (
                ynew_v[pl.ds(16 * i, 16)] * inv_v[pl.ds(16 * i, 16)]
                + z_v[0, pl.ds(16 * i, 16)])
            return carry

          lax.fori_loop(0, RPT // 16, ibody, 0)
        else:

          def rbody(i, carry):
            tot = tmp_v[0, pl.ds(16 * i, 16)]
            for t in range(1, 16):
              tot = tot + tmp_v[t, pl.ds(16 * i, 16)]
            ynew_v[pl.ds(16 * i, 16)] = (
                tot * inv_v[pl.ds(16 * i, 16)]
                + z_v[step, pl.ds(16 * i, 16)])
            return carry

          lax.fori_loop(0, RPT // 16, rbody, 0)

        if k > 0:
          pltpu.sync_copy(ynew_v, sh_y.at[pl.ds(r0, RPT)])
          plsc.subcore_barrier()
          pltpu.sync_copy(sh_y, pval_v)
        else:
          pltpu.sync_copy(ynew_v, out.at[pl.ds(r0, RPT)])

  return sc_horner


def _sc_horner_call(y4, z3, z2, z1, z0, src_t, dst_t):
  return _make_sc_horner()(y4, z3, z2, z1, z0, src_t, dst_t)


def _prep_body(eye_ref, b0, b1, b2, b3, c0, c1, c2, c3, mpack_ref):
  bs = [b0[...], b1[...], b2[...], b3[...]]
  cs = [c0[...], c1[...], c2[...], c3[...]]
  ms = [eye_ref[...], None, None, None, None]
  for l in range(4):
    new = []
    for k in range(5):
      t = None
      if ms[k] is not None:
        t = jnp.dot(ms[k], cs[l], preferred_element_type=jnp.float32)
      if k > 0 and ms[k - 1] is not None:
        tb = jnp.dot(ms[k - 1], bs[l], preferred_element_type=jnp.float32)
        t = tb if t is None else t + tb
      new.append(t)
    ms = new
  cols = [m[:, 0:1] for m in ms]
  cols.append(jnp.zeros((WA, 128 - 5), jnp.float32))
  mpack_ref[...] = jnp.concatenate(cols, axis=1)


def _z_body(x_ref, mp_ref, y4_ref, z3_ref, z2_ref, z1_ref, z0_ref):
  z = jnp.dot(x_ref[...], mp_ref[...], preferred_element_type=jnp.float32)
  y4_ref[...] = z[:, 4]
  z3_ref[...] = z[:, 3]
  z2_ref[...] = z[:, 2]
  z1_ref[...] = z[:, 1]
  z0_ref[...] = z[:, 0]


def _w_spec():
  return pl.BlockSpec((WA, WA), lambda i: (0, 0))


def _prep_call(eye, bs, cs):
  return pl.pallas_call(
      _prep_body,
      grid=(1,),
      in_specs=[_w_spec()] * 9,
      out_specs=pl.BlockSpec((WA, 128), lambda i: (0, 0)),
      out_shape=jax.ShapeDtypeStruct((WA, 128), jnp.float32),
  )(eye, *bs, *cs)


def _z_call(xp2, mpack):
  return pl.pallas_call(
      _z_body,
      grid=(NP // TCB,),
      in_specs=[
          pl.BlockSpec((TCB, WA), lambda i: (i, 0)),
          pl.BlockSpec((WA, 128), lambda i: (0, 0)),
      ],
      out_specs=[pl.BlockSpec((TCB,), lambda i: (i,))] * 5,
      out_shape=[jax.ShapeDtypeStruct((NP,), jnp.float32)] * 5,
  )(xp2, mpack)


def kernel(x, edge_index, Wl0, bl0, Wr0, Wl1, bl1, Wr1, Wl2, bl2, Wr2,
           Wl3, bl3, Wr3):
  f32 = jnp.float32
  ei = edge_index.astype(jnp.int32)
  pad_idx = jnp.full((EP - E,), DUMMY, jnp.int32)
  src_t = jnp.concatenate([ei[0], pad_idx]).reshape(NT, EPT)
  dst_t = jnp.concatenate([ei[1], pad_idx]).reshape(NT, EPT)

  xp2 = jnp.zeros((NP, WA), f32).at[:N, :128].set(x).at[:, ONE_COL].set(1.0)

  wls = [Wl0, Wl1, Wl2, Wl3]
  bls = [bl0, bl1, bl2, bl3]
  wrs = [Wr0, Wr1, Wr2, Wr3]
  bs, cs = [], []
  for l, (din, dout) in enumerate(LDIMS):
    bs.append(jnp.zeros((WA, WA), f32).at[:din, :dout].set(wls[l].T))
    cs.append(
        jnp.zeros((WA, WA), f32)
        .at[:din, :dout].set(wrs[l].T)
        .at[ONE_COL, :dout].set(bls[l])
        .at[ONE_COL, ONE_COL].set(1.0)
    )
  eye = jnp.eye(WA, dtype=f32)

  mpack = _prep_call(eye, bs, cs)
  y4, z3, z2, z1, z0 = _z_call(xp2, mpack)
  y = _sc_horner_call(y4, z3, z2, z1, z0, src_t, dst_t)
  return y[:N].reshape(N, 1)
